# baseline (speedup 1.0000x reference)
import jax
import jax.numpy as jnp
from jax import lax
from jax.experimental import pallas as pl
from jax.experimental.pallas import tpu as pltpu
from jax.experimental.pallas import tpu_sc as plsc

_NUM_CORES = 2
_NUM_SUBCORES = 16
_NW = _NUM_CORES * _NUM_SUBCORES
_V = 1024
_ARCS = 16
_ORDER = 4
_CHUNK = 128
_GROUP = 16


def _sc_body(states, lblt, wt, tot, bwt, btt, unk_in, scores_out, nxt_out,
             idxs, lblbs, wbs, tobs, accb, bwv, sbufs, nbufs, unkv,
             sem, sem2, osems):
    nrows = states.shape[0] // _NW
    nchunks = nrows // _CHUNK
    wid = lax.axis_index("s") * _NUM_CORES + lax.axis_index("c")
    base = wid * nrows

    pltpu.sync_copy(unk_in, unkv)
    zero16f = jnp.zeros((16,), jnp.float32)
    zero16i = jnp.zeros((16,), jnp.int32)
    iota16 = lax.iota(jnp.int32, 16)

    @pl.loop(0, 2 * _V // 16)
    def _zrow(i):
        for bufi in range(2):
            @pl.loop(0, 8)
            def _zr(rr):
                nbufs[bufi][i // (_V // 16), i % (_V // 16) // 8, rr,
                            pl.ds((i % 8) * 16, 16)] = zero16i

    def _undo(nbuf, gold):
        @pl.loop(0, _GROUP)
        def _urow(r):
            rr = gold * _GROUP + r
            b8 = jnp.full((16,), r // 8, jnp.int32)
            r8 = jnp.full((16,), r % 8, jnp.int32)
            for l in range(_ORDER):
                lbl = lblbs[l][rr, :]
                mask = lbl < _V
                lblc = jnp.minimum(lbl, _V - 1)
                plsc.store_scatter(
                    nbuf, [b8, lblc >> 7, r8, lblc & 127], zero16i,
                    mask=mask)

    @pl.loop(0, nchunks)
    def _chunk(ci):
        cbase = base + ci * _CHUNK
        pltpu.sync_copy(states.at[pl.ds(cbase, _CHUNK)], idxs[0])

        arc_cps = []
        for l in range(_ORDER):
            src = idxs[l]
            arc_cps += [
                pltpu.async_copy(lblt.at[src], lblbs[l], sem),
                pltpu.async_copy(wt.at[src], wbs[l], sem),
                pltpu.async_copy(tot.at[src], tobs[l], sem),
            ]
            cps = [pltpu.async_copy(bwt.at[src], bwv, sem2)]
            if l < _ORDER - 1:
                cps.append(pltpu.async_copy(btt.at[src], idxs[l + 1], sem2))
            for c in cps:
                c.wait()
            for i in range(_CHUNK // 16):
                s = pl.ds(i * 16, 16)
                prev = (zero16f if l == 0
                        else accb[pl.ds((l - 1) * _CHUNK + i * 16, 16)])
                accb[pl.ds(l * _CHUNK + i * 16, 16)] = prev + bwv[s]
        for c in arc_cps:
            c.wait()

        unk = unkv[...]

        @pl.loop(0, _CHUNK // (2 * _GROUP))
        def _gpair(gp):
            for bufi in range(2):
                g = gp * 2 + bufi
                sbuf, nbuf, osem = sbufs[bufi], nbufs[bufi], osems[bufi]

                @pl.when(gp > 0)
                def _drain():
                    pltpu.make_async_copy(
                        scores_out.at[pl.ds(0, _GROUP // 8)], sbuf,
                        osem).wait()
                    pltpu.make_async_copy(
                        nxt_out.at[pl.ds(0, _GROUP // 8)], nbuf,
                        osem).wait()
                    _undo(nbuf, (gp - 1) * 2 + bufi)

                @pl.loop(0, _GROUP)
                def _row(r):
                    rr = g * _GROUP + r
                    b8s = r // 8
                    r8s = r % 8
                    b8 = jnp.full((16,), b8s, jnp.int32)
                    r8 = jnp.full((16,), r8s, jnp.int32)

                    def _bcast(lvl):
                        return plsc.load_gather(
                            accb,
                            [jnp.full((16,), lvl * _CHUNK + rr, jnp.int32)])

                    fill = _bcast(_ORDER - 1) + unk
                    for i in range(_V // 16):
                        sbuf[b8s, i // 8, r8s, pl.ds((i % 8) * 16, 16)] = fill
                    for l in range(_ORDER - 1, -1, -1):
                        lbl = lblbs[l][rr, :]
                        wv = wbs[l][rr, :]
                        tv = tobs[l][rr, :]
                        accl = zero16f if l == 0 else _bcast(l - 1)
                        mask = lbl < _V
                        lblc = jnp.minimum(lbl, _V - 1)
                        c8 = lblc >> 7
                        cc = lblc & 127
                        plsc.store_scatter(
                            sbuf, [b8, c8, r8, cc], accl + wv, mask=mask)
                        plsc.store_scatter(
                            nbuf, [b8, c8, r8, cc], tv, mask=mask)

                row0 = cbase + g * _GROUP
                pltpu.async_copy(
                    sbuf, scores_out.at[pl.ds(row0 // 8, _GROUP // 8)], osem)
                pltpu.async_copy(
                    nbuf, nxt_out.at[pl.ds(row0 // 8, _GROUP // 8)], osem)

        for bufi in range(2):
            pltpu.make_async_copy(
                scores_out.at[pl.ds(0, _GROUP // 8)], sbufs[bufi],
                osems[bufi]).wait()
            pltpu.make_async_copy(
                nxt_out.at[pl.ds(0, _GROUP // 8)], nbufs[bufi],
                osems[bufi]).wait()
            _undo(nbufs[bufi], 6 + bufi)


def _retile_body(s_in, n_in, s_out, n_out):
    s_out[...] = s_in[0, 0]
    n_out[...] = n_in[0, 0]


def _body_wrapper(states, lblt, wt, tot, bwt, btt, unk_in, scores_out,
                  nxt_out, i0, i1, i2, i3,
                  l0, l1, l2, l3, w0, w1, w2, w3, t0, t1, t2, t3,
                  accb, bwv, sb0, sb1, nb0, nb1, unkv, sem, sem2,
                  osem0, osem1):
    _sc_body(states, lblt, wt, tot, bwt, btt, unk_in, scores_out, nxt_out,
             [i0, i1, i2, i3], [l0, l1, l2, l3], [w0, w1, w2, w3],
             [t0, t1, t2, t3], accb, bwv, [sb0, sb1], [nb0, nb1], unkv,
             sem, sem2, [osem0, osem1])


def kernel(states, arc_labels, arc_weights, arc_to, backoff_weights,
           backoff_to, unk_prob):
    b = states.shape[0]
    unk16 = jnp.broadcast_to(unk_prob.astype(jnp.float32), (16,))
    mesh = plsc.VectorSubcoreMesh(
        core_axis_name="c", subcore_axis_name="s",
        num_cores=_NUM_CORES, num_subcores=_NUM_SUBCORES)
    xv = lambda: pltpu.VMEM((_CHUNK,), jnp.int32)
    iv = lambda: pltpu.VMEM((_CHUNK, _ARCS), jnp.int32)
    fv = lambda: pltpu.VMEM((_CHUNK, _ARCS), jnp.float32)
    ob = lambda dt: pltpu.VMEM((_GROUP // 8, _V // 128, 8, 128), dt)
    o4 = lambda dt: jax.ShapeDtypeStruct((b // 8, _V // 128, 8, 128), dt)
    scores4, nxt4 = pl.kernel(
        _body_wrapper,
        out_type=[o4(jnp.float32), o4(jnp.int32)],
        mesh=mesh,
        compiler_params=pltpu.CompilerParams(
            needs_layout_passes=False, use_tc_tiling_on_sc=False),
        scratch_types=[
            xv(), xv(), xv(), xv(),
            iv(), iv(), iv(), iv(),
            fv(), fv(), fv(), fv(),
            iv(), iv(), iv(), iv(),
            pltpu.VMEM((_ORDER * _CHUNK,), jnp.float32),
            pltpu.VMEM((_CHUNK,), jnp.float32),
            ob(jnp.float32), ob(jnp.float32),
            ob(jnp.int32), ob(jnp.int32),
            pltpu.VMEM((16,), jnp.float32),
            pltpu.SemaphoreType.DMA,
            pltpu.SemaphoreType.DMA,
            pltpu.SemaphoreType.DMA,
            pltpu.SemaphoreType.DMA,
        ],
    )(states, arc_labels, arc_weights, arc_to, backoff_weights,
      backoff_to, unk16)
    scores, nxt = pl.pallas_call(
        _retile_body,
        grid=(b // 8, _V // 128),
        in_specs=[
            pl.BlockSpec((1, 1, 8, 128), lambda i, j: (i, j, 0, 0)),
            pl.BlockSpec((1, 1, 8, 128), lambda i, j: (i, j, 0, 0)),
        ],
        out_specs=[
            pl.BlockSpec((8, 128), lambda i, j: (i, j)),
            pl.BlockSpec((8, 128), lambda i, j: (i, j)),
        ],
        out_shape=[
            jax.ShapeDtypeStruct((b, _V), jnp.float32),
            jax.ShapeDtypeStruct((b, _V), jnp.int32),
        ],
    )(scores4, nxt4)
    return scores, nxt


# speedup vs baseline: 5.3961x; 5.3961x over previous
import jax
import jax.numpy as jnp
from jax import lax
from jax.experimental import pallas as pl
from jax.experimental.pallas import tpu as pltpu
from jax.experimental.pallas import tpu_sc as plsc

_NUM_CORES = 2
_NUM_SUBCORES = 16
_NW = _NUM_CORES * _NUM_SUBCORES
_V = 1024
_ARCS = 16
_ORDER = 4
_CHUNK = 128
_GROUP = 16


def _sc_body(states, lblt, wt, tot, bwt, btt, unk_in, scores_out, nxt_out,
             idxs, lblbs, wbs, tobs, accb, bwv, sbufs, nbufs, unkv,
             sem, sem2, osems):
    nrows = states.shape[0] // _NW
    nchunks = nrows // _CHUNK
    wid = lax.axis_index("s") * _NUM_CORES + lax.axis_index("c")
    base = wid * nrows

    pltpu.sync_copy(unk_in, unkv)
    zero16f = jnp.zeros((16,), jnp.float32)
    zero16i = jnp.zeros((16,), jnp.int32)
    iota16 = lax.iota(jnp.int32, 16)

    @pl.loop(0, 2 * _V // 16)
    def _zrow(i):
        for bufi in range(2):
            @pl.loop(0, 8)
            def _zr(rr):
                nbufs[bufi][i // (_V // 16), i % (_V // 16) // 8, rr,
                            pl.ds((i % 8) * 16, 16)] = zero16i

    def _undo(nbuf, gold):
        @pl.loop(0, _GROUP)
        def _urow(r):
            rr = gold * _GROUP + r
            b8 = jnp.full((16,), r // 8, jnp.int32)
            r8 = jnp.full((16,), r % 8, jnp.int32)
            for l in range(_ORDER):
                lbl = lblbs[l][rr, :]
                mask = lbl < _V
                lblc = jnp.minimum(lbl, _V - 1)
                plsc.store_scatter(
                    nbuf, [b8, lblc >> 7, r8, lblc & 127], zero16i,
                    mask=mask)

    @pl.loop(0, nchunks)
    def _chunk(ci):
        cbase = base + ci * _CHUNK
        pltpu.sync_copy(states.at[pl.ds(cbase, _CHUNK)], idxs[0])

        arc_cps = []
        for l in range(_ORDER):
            src = idxs[l]
            arc_cps += [
                pltpu.async_copy(lblt.at[src], lblbs[l], sem),
                pltpu.async_copy(wt.at[src], wbs[l], sem),
                pltpu.async_copy(tot.at[src], tobs[l], sem),
            ]
            cps = [pltpu.async_copy(bwt.at[src], bwv, sem2)]
            if l < _ORDER - 1:
                cps.append(pltpu.async_copy(btt.at[src], idxs[l + 1], sem2))
            for c in cps:
                c.wait()
            for i in range(_CHUNK // 16):
                s = pl.ds(i * 16, 16)
                prev = (zero16f if l == 0
                        else accb[pl.ds((l - 1) * _CHUNK + i * 16, 16)])
                accb[pl.ds(l * _CHUNK + i * 16, 16)] = prev + bwv[s]
        for c in arc_cps:
            c.wait()

        unk = unkv[...]

        @pl.loop(0, _CHUNK // (2 * _GROUP))
        def _gpair(gp):
            for bufi in range(2):
                g = gp * 2 + bufi
                sbuf, nbuf, osem = sbufs[bufi], nbufs[bufi], osems[bufi]

                @pl.when(gp > 0)
                def _drain():
                    pltpu.make_async_copy(
                        scores_out.at[pl.ds(0, _GROUP // 8)], sbuf,
                        osem).wait()
                    pltpu.make_async_copy(
                        nxt_out.at[pl.ds(0, _GROUP // 8)], nbuf,
                        osem).wait()
                    _undo(nbuf, (gp - 1) * 2 + bufi)

                @pl.loop(0, _GROUP)
                def _row(r):
                    rr = g * _GROUP + r
                    b8s = r // 8
                    r8s = r % 8
                    b8 = jnp.full((16,), b8s, jnp.int32)
                    r8 = jnp.full((16,), r8s, jnp.int32)

                    def _bcast(lvl):
                        return plsc.load_gather(
                            accb,
                            [jnp.full((16,), lvl * _CHUNK + rr, jnp.int32)])

                    fill = _bcast(_ORDER - 1) + unk
                    for i in range(_V // 16):
                        sbuf[b8s, i // 8, r8s, pl.ds((i % 8) * 16, 16)] = fill
                    for l in range(_ORDER - 1, -1, -1):
                        lbl = lblbs[l][rr, :]
                        wv = wbs[l][rr, :]
                        tv = tobs[l][rr, :]
                        accl = zero16f if l == 0 else _bcast(l - 1)
                        mask = lbl < _V
                        lblc = jnp.minimum(lbl, _V - 1)
                        c8 = lblc >> 7
                        cc = lblc & 127
                        plsc.store_scatter(
                            sbuf, [b8, c8, r8, cc], accl + wv, mask=mask)
                        plsc.store_scatter(
                            nbuf, [b8, c8, r8, cc], tv, mask=mask)

                row0 = cbase + g * _GROUP
                pltpu.async_copy(
                    sbuf, scores_out.at[pl.ds(row0 // 8, _GROUP // 8)], osem)
                pltpu.async_copy(
                    nbuf, nxt_out.at[pl.ds(row0 // 8, _GROUP // 8)], osem)

        for bufi in range(2):
            pltpu.make_async_copy(
                scores_out.at[pl.ds(0, _GROUP // 8)], sbufs[bufi],
                osems[bufi]).wait()
            pltpu.make_async_copy(
                nxt_out.at[pl.ds(0, _GROUP // 8)], nbufs[bufi],
                osems[bufi]).wait()
            _undo(nbufs[bufi], 6 + bufi)


def _retile_body(s_in, n_in, s_out, n_out):
    s_out[...] = s_in[...].reshape(s_out.shape)
    n_out[...] = n_in[...].reshape(n_out.shape)


def _body_wrapper(states, lblt, wt, tot, bwt, btt, unk_in, scores_out,
                  nxt_out, i0, i1, i2, i3,
                  l0, l1, l2, l3, w0, w1, w2, w3, t0, t1, t2, t3,
                  accb, bwv, sb0, sb1, nb0, nb1, unkv, sem, sem2,
                  osem0, osem1):
    _sc_body(states, lblt, wt, tot, bwt, btt, unk_in, scores_out, nxt_out,
             [i0, i1, i2, i3], [l0, l1, l2, l3], [w0, w1, w2, w3],
             [t0, t1, t2, t3], accb, bwv, [sb0, sb1], [nb0, nb1], unkv,
             sem, sem2, [osem0, osem1])


def kernel(states, arc_labels, arc_weights, arc_to, backoff_weights,
           backoff_to, unk_prob):
    b = states.shape[0]
    unk16 = jnp.broadcast_to(unk_prob.astype(jnp.float32), (16,))
    mesh = plsc.VectorSubcoreMesh(
        core_axis_name="c", subcore_axis_name="s",
        num_cores=_NUM_CORES, num_subcores=_NUM_SUBCORES)
    xv = lambda: pltpu.VMEM((_CHUNK,), jnp.int32)
    iv = lambda: pltpu.VMEM((_CHUNK, _ARCS), jnp.int32)
    fv = lambda: pltpu.VMEM((_CHUNK, _ARCS), jnp.float32)
    ob = lambda dt: pltpu.VMEM((_GROUP // 8, _V // 128, 8, 128), dt)
    o4 = lambda dt: jax.ShapeDtypeStruct((b // 8, _V // 128, 8, 128), dt)
    scores4, nxt4 = pl.kernel(
        _body_wrapper,
        out_type=[o4(jnp.float32), o4(jnp.int32)],
        mesh=mesh,
        compiler_params=pltpu.CompilerParams(
            needs_layout_passes=False, use_tc_tiling_on_sc=False),
        scratch_types=[
            xv(), xv(), xv(), xv(),
            iv(), iv(), iv(), iv(),
            fv(), fv(), fv(), fv(),
            iv(), iv(), iv(), iv(),
            pltpu.VMEM((_ORDER * _CHUNK,), jnp.float32),
            pltpu.VMEM((_CHUNK,), jnp.float32),
            ob(jnp.float32), ob(jnp.float32),
            ob(jnp.int32), ob(jnp.int32),
            pltpu.VMEM((16,), jnp.float32),
            pltpu.SemaphoreType.DMA,
            pltpu.SemaphoreType.DMA,
            pltpu.SemaphoreType.DMA,
            pltpu.SemaphoreType.DMA,
        ],
    )(states, arc_labels, arc_weights, arc_to, backoff_weights,
      backoff_to, unk16)
    scores, nxt = pl.pallas_call(
        _retile_body,
        grid=(b // 64, _V // 128),
        in_specs=[
            pl.BlockSpec((8, 1, 8, 128), lambda i, j: (i, j, 0, 0)),
            pl.BlockSpec((8, 1, 8, 128), lambda i, j: (i, j, 0, 0)),
        ],
        out_specs=[
            pl.BlockSpec((64, 128), lambda i, j: (i, j)),
            pl.BlockSpec((64, 128), lambda i, j: (i, j)),
        ],
        out_shape=[
            jax.ShapeDtypeStruct((b, _V), jnp.float32),
            jax.ShapeDtypeStruct((b, _V), jnp.int32),
        ],
    )(scores4, nxt4)
    return scores, nxt


# speedup vs baseline: 20.2607x; 3.7547x over previous
import jax
import jax.numpy as jnp
from jax import lax
from jax.experimental import pallas as pl
from jax.experimental.pallas import tpu as pltpu
from jax.experimental.pallas import tpu_sc as plsc

_NUM_CORES = 2
_NUM_SUBCORES = 16
_NW = _NUM_CORES * _NUM_SUBCORES
_V = 1024
_ARCS = 16
_ORDER = 4
_CHUNK = 128
_GROUP = 16


def _sc_body(states, lblt, wt, tot, bwt, btt, unk_in, scores_out, nxt_out,
             idxs, lblbs, wbs, tobs, accb, bwv, sbufs, nbufs, unkv,
             sem, sem2, osems):
    nrows = states.shape[0] // _NW
    nchunks = nrows // _CHUNK
    wid = lax.axis_index("s") * _NUM_CORES + lax.axis_index("c")
    base = wid * nrows

    pltpu.sync_copy(unk_in, unkv)
    zero16f = jnp.zeros((16,), jnp.float32)
    zero16i = jnp.zeros((16,), jnp.int32)
    iota16 = lax.iota(jnp.int32, 16)

    @pl.loop(0, 2 * _V // 16)
    def _zrow(i):
        for bufi in range(2):
            @pl.loop(0, 8)
            def _zr(rr):
                nbufs[bufi][i // (_V // 16), i % (_V // 16) // 8, rr,
                            pl.ds((i % 8) * 16, 16)] = zero16i

    def _undo(nbuf, gold):
        @pl.loop(0, _GROUP)
        def _urow(r):
            rr = gold * _GROUP + r
            b8 = jnp.full((16,), r // 8, jnp.int32)
            r8 = jnp.full((16,), r % 8, jnp.int32)
            for l in range(_ORDER):
                lbl = lblbs[l][rr, :]
                mask = lbl < _V
                lblc = jnp.minimum(lbl, _V - 1)
                plsc.store_scatter(
                    nbuf, [b8, lblc >> 7, r8, lblc & 127], zero16i,
                    mask=mask)

    @pl.loop(0, nchunks)
    def _chunk(ci):
        cbase = base + ci * _CHUNK
        pltpu.sync_copy(states.at[pl.ds(cbase, _CHUNK)], idxs[0])

        arc_cps = []
        for l in range(_ORDER):
            src = idxs[l]
            arc_cps += [
                pltpu.async_copy(lblt.at[src], lblbs[l], sem),
                pltpu.async_copy(wt.at[src], wbs[l], sem),
                pltpu.async_copy(tot.at[src], tobs[l], sem),
            ]
            cps = [pltpu.async_copy(bwt.at[src], bwv, sem2)]
            if l < _ORDER - 1:
                cps.append(pltpu.async_copy(btt.at[src], idxs[l + 1], sem2))
            for c in cps:
                c.wait()
            for i in range(_CHUNK // 16):
                s = pl.ds(i * 16, 16)
                prev = (zero16f if l == 0
                        else accb[pl.ds((l - 1) * _CHUNK + i * 16, 16)])
                accb[pl.ds(l * _CHUNK + i * 16, 16)] = prev + bwv[s]
        for c in arc_cps:
            c.wait()

        unk = unkv[...]

        @pl.loop(0, _CHUNK // (2 * _GROUP))
        def _gpair(gp):
            for bufi in range(2):
                g = gp * 2 + bufi
                sbuf, nbuf, osem = sbufs[bufi], nbufs[bufi], osems[bufi]

                @pl.when(gp > 0)
                def _drain():
                    pltpu.make_async_copy(
                        scores_out.at[pl.ds(0, _GROUP // 8)], sbuf,
                        osem).wait()
                    pltpu.make_async_copy(
                        nxt_out.at[pl.ds(0, _GROUP // 8)], nbuf,
                        osem).wait()
                    _undo(nbuf, (gp - 1) * 2 + bufi)

                @pl.loop(0, _GROUP)
                def _row(r):
                    rr = g * _GROUP + r
                    b8s = r // 8
                    r8s = r % 8
                    b8 = jnp.full((16,), b8s, jnp.int32)
                    r8 = jnp.full((16,), r8s, jnp.int32)

                    def _bcast(lvl):
                        return plsc.load_gather(
                            accb,
                            [jnp.full((16,), lvl * _CHUNK + rr, jnp.int32)])

                    fill = _bcast(_ORDER - 1) + unk
                    for i in range(_V // 16):
                        sbuf[b8s, i // 8, r8s, pl.ds((i % 8) * 16, 16)] = fill
                    for l in range(_ORDER - 1, -1, -1):
                        lbl = lblbs[l][rr, :]
                        wv = wbs[l][rr, :]
                        tv = tobs[l][rr, :]
                        accl = zero16f if l == 0 else _bcast(l - 1)
                        mask = lbl < _V
                        lblc = jnp.minimum(lbl, _V - 1)
                        c8 = lblc >> 7
                        cc = lblc & 127
                        plsc.store_scatter(
                            sbuf, [b8, c8, r8, cc], accl + wv, mask=mask)
                        plsc.store_scatter(
                            nbuf, [b8, c8, r8, cc], tv, mask=mask)

                row0 = cbase + g * _GROUP
                pltpu.async_copy(
                    sbuf, scores_out.at[pl.ds(row0 // 8, _GROUP // 8)], osem)
                pltpu.async_copy(
                    nbuf, nxt_out.at[pl.ds(row0 // 8, _GROUP // 8)], osem)

        for bufi in range(2):
            pltpu.make_async_copy(
                scores_out.at[pl.ds(0, _GROUP // 8)], sbufs[bufi],
                osems[bufi]).wait()
            pltpu.make_async_copy(
                nxt_out.at[pl.ds(0, _GROUP // 8)], nbufs[bufi],
                osems[bufi]).wait()
            _undo(nbufs[bufi], 6 + bufi)


def _body_wrapper(states, lblt, wt, tot, bwt, btt, unk_in, scores_out,
                  nxt_out, i0, i1, i2, i3,
                  l0, l1, l2, l3, w0, w1, w2, w3, t0, t1, t2, t3,
                  accb, bwv, sb0, sb1, nb0, nb1, unkv, sem, sem2,
                  osem0, osem1):
    _sc_body(states, lblt, wt, tot, bwt, btt, unk_in, scores_out, nxt_out,
             [i0, i1, i2, i3], [l0, l1, l2, l3], [w0, w1, w2, w3],
             [t0, t1, t2, t3], accb, bwv, [sb0, sb1], [nb0, nb1], unkv,
             sem, sem2, [osem0, osem1])


def kernel(states, arc_labels, arc_weights, arc_to, backoff_weights,
           backoff_to, unk_prob):
    b = states.shape[0]
    unk16 = jnp.broadcast_to(unk_prob.astype(jnp.float32), (16,))
    mesh = plsc.VectorSubcoreMesh(
        core_axis_name="c", subcore_axis_name="s",
        num_cores=_NUM_CORES, num_subcores=_NUM_SUBCORES)
    xv = lambda: pltpu.VMEM((_CHUNK,), jnp.int32)
    iv = lambda: pltpu.VMEM((_CHUNK, _ARCS), jnp.int32)
    fv = lambda: pltpu.VMEM((_CHUNK, _ARCS), jnp.float32)
    ob = lambda dt: pltpu.VMEM((_GROUP // 8, _V // 128, 8, 128), dt)
    o4 = lambda dt: jax.ShapeDtypeStruct((b // 8, _V // 128, 8, 128), dt)
    scores4, nxt4 = pl.kernel(
        _body_wrapper,
        out_type=[o4(jnp.float32), o4(jnp.int32)],
        mesh=mesh,
        compiler_params=pltpu.CompilerParams(
            needs_layout_passes=False, use_tc_tiling_on_sc=False),
        scratch_types=[
            xv(), xv(), xv(), xv(),
            iv(), iv(), iv(), iv(),
            fv(), fv(), fv(), fv(),
            iv(), iv(), iv(), iv(),
            pltpu.VMEM((_ORDER * _CHUNK,), jnp.float32),
            pltpu.VMEM((_CHUNK,), jnp.float32),
            ob(jnp.float32), ob(jnp.float32),
            ob(jnp.int32), ob(jnp.int32),
            pltpu.VMEM((16,), jnp.float32),
            pltpu.SemaphoreType.DMA,
            pltpu.SemaphoreType.DMA,
            pltpu.SemaphoreType.DMA,
            pltpu.SemaphoreType.DMA,
        ],
    )(states, arc_labels, arc_weights, arc_to, backoff_weights,
      backoff_to, unk16)
    scores = scores4.transpose(0, 2, 1, 3).reshape(b, _V)
    nxt = nxt4.transpose(0, 2, 1, 3).reshape(b, _V)
    return scores, nxt
